# Initial kernel scaffold; baseline (speedup 1.0000x reference)
import jax
import jax.numpy as jnp
from jax.experimental import pallas as pl


def kernel(features, prototypes):
    raise NotImplementedError("write your pallas kernel here")


# speedup vs baseline: 4.4079x; 4.4079x over previous
import jax
import jax.numpy as jnp
from jax.experimental import pallas as pl
from jax.experimental.pallas import tpu as pltpu

TAU = 0.2
EPS = 1e-8


def _assign_kernel(f_ref, p_ref, logits_ref, probs_ref):
    f = f_ref[...]
    norm = jnp.sqrt(jnp.sum(f * f, axis=-1, keepdims=True))
    nf = f / jnp.maximum(norm, EPS)
    logits = jax.lax.dot_general(
        nf, p_ref[...], (((1,), (1,)), ((), ())),
        preferred_element_type=jnp.float32,
    ) * (1.0 / max(TAU, EPS))
    logits_ref[...] = logits
    m = jnp.max(logits, axis=-1, keepdims=True)
    e = jnp.exp(logits - m)
    s = jnp.sum(e, axis=-1, keepdims=True)
    probs_ref[...] = e / s


def kernel(features, prototypes):
    n_tokens, fdim = features.shape
    n_proto = prototypes.shape[0]
    tm = 256
    grid = (n_tokens // tm,)
    logits, probs = pl.pallas_call(
        _assign_kernel,
        grid=grid,
        in_specs=[
            pl.BlockSpec((tm, fdim), lambda i: (i, 0)),
            pl.BlockSpec((n_proto, fdim), lambda i: (0, 0)),
        ],
        out_specs=[
            pl.BlockSpec((tm, n_proto), lambda i: (i, 0)),
            pl.BlockSpec((tm, n_proto), lambda i: (i, 0)),
        ],
        out_shape=[
            jax.ShapeDtypeStruct((n_tokens, n_proto), jnp.float32),
            jax.ShapeDtypeStruct((n_tokens, n_proto), jnp.float32),
        ],
        compiler_params=pltpu.CompilerParams(
            dimension_semantics=("arbitrary",),
        ),
    )(features, prototypes)
    return (logits, probs)


# speedup vs baseline: 4.4091x; 1.0003x over previous
import jax
import jax.numpy as jnp
from jax.experimental import pallas as pl
from jax.experimental.pallas import tpu as pltpu

TAU = 0.2
EPS = 1e-8


def _assign_kernel(f_ref, p_ref, logits_ref, probs_ref):
    f = f_ref[...]
    norm = jnp.sqrt(jnp.sum(f * f, axis=-1, keepdims=True))
    nf = f / jnp.maximum(norm, EPS)
    logits = jax.lax.dot_general(
        nf, p_ref[...], (((1,), (1,)), ((), ())),
        preferred_element_type=jnp.float32,
    ) * (1.0 / max(TAU, EPS))
    logits_ref[...] = logits
    m = jnp.max(logits, axis=-1, keepdims=True)
    e = jnp.exp(logits - m)
    s = jnp.sum(e, axis=-1, keepdims=True)
    probs_ref[...] = e / s


def kernel(features, prototypes):
    n_tokens, fdim = features.shape
    n_proto = prototypes.shape[0]
    tm = 256
    grid = (n_tokens // tm,)
    logits, probs = pl.pallas_call(
        _assign_kernel,
        grid=grid,
        in_specs=[
            pl.BlockSpec((tm, fdim), lambda i: (i, 0)),
            pl.BlockSpec((n_proto, fdim), lambda i: (0, 0)),
        ],
        out_specs=[
            pl.BlockSpec((tm, n_proto), lambda i: (i, 0)),
            pl.BlockSpec((tm, n_proto), lambda i: (i, 0)),
        ],
        out_shape=[
            jax.ShapeDtypeStruct((n_tokens, n_proto), jnp.float32),
            jax.ShapeDtypeStruct((n_tokens, n_proto), jnp.float32),
        ],
        compiler_params=pltpu.CompilerParams(
            dimension_semantics=("parallel",),
        ),
    )(features, prototypes)
    return (logits, probs)
